# baseline (speedup 1.0000x reference)
import functools

import numpy as np
import jax
import jax.numpy as jnp
from jax import lax
from jax.experimental import pallas as pl
from jax.experimental.pallas import tpu as pltpu
from jax.experimental.pallas import tpu_sc as plsc

_NF = 26
_ROWS_PER_FIELD = 100000
_BATCH = 16384
_B = _BATCH * _NF
_D = 32
_NW = 32
_BPW = _B // _NW
_C = 832
_NCHUNK = _BPW // _C

_FIELD_OFFS = np.tile(
    np.arange(_NF, dtype=np.int32) * _ROWS_PER_FIELD, _BPW // _NF
)

_mesh = plsc.VectorSubcoreMesh(core_axis_name="c", subcore_axis_name="s")


@functools.partial(
    pl.kernel,
    out_type=jax.ShapeDtypeStruct((_B, _D), jnp.float32),
    mesh=_mesh,
    compiler_params=pltpu.CompilerParams(use_tc_tiling_on_sc=False),
    scratch_types=[
        pltpu.VMEM((_BPW,), jnp.int32),
        pltpu.VMEM((_BPW,), jnp.int32),
        pltpu.VMEM((_C, _D), jnp.float32),
        pltpu.VMEM((_C, _D), jnp.float32),
        pltpu.SemaphoreType.DMA,
        pltpu.SemaphoreType.DMA,
    ],
)
def _embed_gather(idx_hbm, offs_hbm, table_hbm, out_hbm,
                  idx_v, offs_v, rows0, rows1, gsem, osem):
    wid = lax.axis_index("s") * 2 + lax.axis_index("c")
    base = wid * _BPW

    pltpu.sync_copy(idx_hbm.at[pl.ds(base, _BPW)], idx_v)
    pltpu.sync_copy(offs_hbm, offs_v)

    def _add_offsets(i, carry):
        s = pl.ds(i * 16, 16)
        idx_v[s] = idx_v[s] + offs_v[s]
        return carry

    lax.fori_loop(0, _BPW // 16, _add_offsets, 0)

    bufs = (rows0, rows1)
    for g in range(_NCHUNK):
        buf = bufs[g % 2]
        pltpu.async_copy(
            table_hbm.at[idx_v.at[pl.ds(g * _C, _C)]], buf, gsem
        ).wait()
        pltpu.sync_copy(buf, out_hbm.at[pl.ds(base + g * _C, _C)])


def kernel(x, table):
    idx = x.reshape(_B).astype(jnp.int32)
    offs = jnp.asarray(_FIELD_OFFS)
    out = _embed_gather(idx, offs, table)
    return out.reshape(_BATCH, _NF, _D)


# speedup vs baseline: 1.0072x; 1.0072x over previous
import functools

import numpy as np
import jax
import jax.numpy as jnp
from jax import lax
from jax.experimental import pallas as pl
from jax.experimental.pallas import tpu as pltpu
from jax.experimental.pallas import tpu_sc as plsc

_NF = 26
_ROWS_PER_FIELD = 100000
_BATCH = 16384
_B = _BATCH * _NF
_D = 32
_NW = 32
_BPW = _B // _NW
_C = 832
_NCHUNK = _BPW // _C

_FIELD_OFFS = np.tile(
    np.arange(_NF, dtype=np.int32) * _ROWS_PER_FIELD, _BPW // _NF
)

_mesh = plsc.VectorSubcoreMesh(core_axis_name="c", subcore_axis_name="s")


@functools.partial(
    pl.kernel,
    out_type=jax.ShapeDtypeStruct((_B, _D), jnp.float32),
    mesh=_mesh,
    compiler_params=pltpu.CompilerParams(use_tc_tiling_on_sc=False),
    scratch_types=[
        pltpu.VMEM((_BPW,), jnp.int32),
        pltpu.VMEM((_BPW,), jnp.int32),
        pltpu.VMEM((_C, _D), jnp.float32),
        pltpu.VMEM((_C, _D), jnp.float32),
        pltpu.SemaphoreType.DMA,
        pltpu.SemaphoreType.DMA,
        pltpu.SemaphoreType.DMA,
        pltpu.SemaphoreType.DMA,
    ],
)
def _embed_gather(idx_hbm, offs_hbm, table_hbm, out_hbm,
                  idx_v, offs_v, rows0, rows1, gsem0, gsem1, osem0, osem1):
    wid = lax.axis_index("s") * 2 + lax.axis_index("c")
    base = wid * _BPW

    pltpu.sync_copy(idx_hbm.at[pl.ds(base, _BPW)], idx_v)
    pltpu.sync_copy(offs_hbm, offs_v)

    def _add_offsets(i, carry):
        s = pl.ds(i * 16, 16)
        idx_v[s] = idx_v[s] + offs_v[s]
        return carry

    lax.fori_loop(0, _BPW // 16, _add_offsets, 0)

    bufs = (rows0, rows1)
    gsems = (gsem0, gsem1)
    osems = (osem0, osem1)

    def _start_gather(g):
        return pltpu.async_copy(
            table_hbm.at[idx_v.at[pl.ds(g * _C, _C)]], bufs[g % 2], gsems[g % 2]
        )

    def _start_out(g):
        return pltpu.async_copy(
            bufs[g % 2], out_hbm.at[pl.ds(base + g * _C, _C)], osems[g % 2]
        )

    gcp = [None, None]
    ocp = [None, None]
    gcp[0] = _start_gather(0)
    for g in range(1, _NCHUNK + 1):
        if g < _NCHUNK:
            if ocp[g % 2] is not None:
                ocp[g % 2].wait()
            gcp[g % 2] = _start_gather(g)
        gcp[(g - 1) % 2].wait()
        ocp[(g - 1) % 2] = _start_out(g - 1)
    ocp[(_NCHUNK - 2) % 2].wait()
    ocp[(_NCHUNK - 1) % 2].wait()


def kernel(x, table):
    idx = x.reshape(_B).astype(jnp.int32)
    offs = jnp.asarray(_FIELD_OFFS)
    out = _embed_gather(idx, offs, table)
    return out.reshape(_BATCH, _NF, _D)
